# baseline (speedup 1.0000x reference)
import functools

import jax
import jax.numpy as jnp
from jax import lax
from jax.experimental import pallas as pl
from jax.experimental.pallas import tpu as pltpu
from jax.experimental.pallas import tpu_sc as plsc

N = 320000
S = 10000
D = 128
NC = 2
NS = 16
NW = NC * NS
S_PAD = NW * 320
SEG = 320
ACC_ROWS = SEG + 8
CHUNK = 80
NBUF = 4


def _body(x_hbm, idx_hbm, starts_hbm, out_hbm,
          xb0, xb1, xb2, xb3, ib0, ib1, ib2, ib3, startsbuf, ssums, ends,
          sx0, sx1, sx2, sx3, si0, si1, si2, si3, ss0, ss1, ss2, ss3):
    c = lax.axis_index("c")
    s = lax.axis_index("s")
    wid = s * NC + c

    XB = [xb0, xb1, xb2, xb3]
    IB = [ib0, ib1, ib2, ib3]
    SX = [sx0, sx1, sx2, sx3]
    SI = [si0, si1, si2, si3]
    SS = [ss0, ss1, ss2, ss3]

    zero16 = jnp.zeros((16,), jnp.float32)
    iota16 = lax.iota(jnp.int32, 16)
    sbase = s * ACC_ROWS

    def frow(i, _):
        for j in range(8):
            xb0[i, pl.ds(16 * j, 16)] = zero16
        return 0
    lax.fori_loop(0, CHUNK, frow, 0)
    for k in range(4):
        pltpu.sync_copy(xb0, ssums.at[pl.ds(sbase + k * CHUNK, CHUNK)])
    pltpu.sync_copy(xb0.at[pl.ds(0, 8)], ssums.at[pl.ds(sbase + SEG, 8)])

    def erow(i, _):
        ends[i] = 0
        return 0
    lax.fori_loop(0, SEG + 8, erow, 0)

    pltpu.sync_copy(starts_hbm, startsbuf)
    sv = startsbuf[pl.ds(wid, 16)]
    start = sv[0]
    end = sv[1]
    astart = (start // 8) * 8
    nwin = (end - astart + (CHUNK - 1)) // CHUNK
    nloops = (nwin + 3) // 4

    def woff(ci):
        return pl.multiple_of(
            jnp.minimum(astart + ci * CHUNK, N - CHUNK), 8)

    def dma_start(ci, b):
        off = woff(ci)
        pltpu.async_copy(x_hbm.at[pl.ds(off, CHUNK)], XB[b], SX[b])
        pltpu.async_copy(idx_hbm.at[pl.ds(off, CHUNK)], IB[b], SI[b])

    def dma_wait(ci, b):
        off = woff(ci)
        pltpu.make_async_copy(x_hbm.at[pl.ds(off, CHUNK)], XB[b], SX[b]).wait()
        pltpu.make_async_copy(idx_hbm.at[pl.ds(off, CHUNK)], IB[b], SI[b]).wait()

    def transform(ci, b):
        off = woff(ci)
        lo = jnp.maximum(start, astart + ci * CHUNK)
        hi = jnp.minimum(end, astart + ci * CHUNK + CHUNK)
        ib = IB[b]
        for j in range(CHUNK // 16):
            iv = ib[pl.ds(16 * j, 16)]
            rows = off + 16 * j + iota16
            valid = (rows >= lo) & (rows < hi)
            tlv = jnp.where(valid, iv - SEG * wid, SEG)
            ib[pl.ds(16 * j, 16)] = tlv + sbase
            for k in range(16):
                ends[tlv[k]] = off + (16 * j + k + 1)

    def scat_start(b):
        pltpu.async_copy(XB[b], ssums.at[IB[b]], SS[b], add=True)

    def scat_wait(b):
        pltpu.make_async_copy(x_hbm.at[pl.ds(0, CHUNK)], XB[b], SS[b]).wait()

    def win_step(ci, b, refill):
        dma_wait(ci, b)
        transform(ci, b)
        scat_start(b)
        b2 = (b + 2) % NBUF
        if refill:
            scat_wait(b2)
            dma_start(ci + 2, b2)

    dma_start(0, 0)
    dma_start(1, 1)
    dma_wait(0, 0)
    transform(0, 0)
    scat_start(0)
    dma_start(2, 2)
    dma_wait(1, 1)
    transform(1, 1)
    scat_start(1)
    dma_start(3, 3)

    def quad(q, _):
        ci = 4 * q + 2
        win_step(ci, 2, True)
        win_step(ci + 1, 3, True)
        win_step(ci + 2, 0, True)
        win_step(ci + 3, 1, True)
        return 0
    lax.fori_loop(0, nloops, quad, 0)

    last = 4 * nloops + 1
    scat_wait(0)
    scat_wait(1)
    dma_wait(last + 1, 2)
    dma_wait(last + 2, 3)

    def divide_chunk(k, pm_in):
        pltpu.sync_copy(ssums.at[pl.ds(sbase + k * CHUNK, CHUNK)], xb1)

        def drow(i, pm):
            e = ends[k * CHUNK + i]
            pm_new = jnp.maximum(pm, e)
            cntf = (pm_new - pm).astype(jnp.float32)
            inv16 = 1.0 / jnp.maximum(jnp.broadcast_to(cntf, (16,)), 1.0)
            for j in range(8):
                sl = pl.ds(16 * j, 16)
                xb1[i, sl] = xb1[i, sl] * inv16
            return pm_new
        pm_out = lax.fori_loop(0, CHUNK, drow, pm_in)

        @pl.when(wid * SEG + k * CHUNK < S)
        def _():
            pltpu.sync_copy(xb1, out_hbm.at[pl.ds(wid * SEG + k * CHUNK, CHUNK)])
        return pm_out

    lax.fori_loop(0, 4, divide_chunk, start)


_segmean = pl.kernel(
    _body,
    out_type=jax.ShapeDtypeStruct((S, D), jnp.float32),
    mesh=plsc.VectorSubcoreMesh(core_axis_name="c", subcore_axis_name="s"),
    scratch_types=(
        [pltpu.VMEM((CHUNK, D), jnp.float32) for _ in range(NBUF)]
        + [pltpu.VMEM((CHUNK,), jnp.int32) for _ in range(NBUF)]
        + [pltpu.VMEM((48,), jnp.int32)]
        + [pltpu.VMEM_SHARED((NS * ACC_ROWS, D), jnp.float32)]
        + [pltpu.SMEM((SEG + 8,), jnp.int32)]
        + [pltpu.SemaphoreType.DMA for _ in range(3 * NBUF)]
    ),
)


def kernel(x, index):
    bounds = jnp.arange(0, S_PAD + 1, SEG, dtype=jnp.int32)
    starts = jnp.sum(index[None, :] < bounds[:, None], axis=1, dtype=jnp.int32)
    starts = jnp.pad(starts, (0, 48 - starts.shape[0]))
    return _segmean(x, index, starts)
